# baseline (speedup 1.0000x reference)
import functools

import jax
import jax.numpy as jnp
from jax import lax
from jax.experimental import pallas as pl
from jax.experimental.pallas import tpu as pltpu
from jax.experimental.pallas import tpu_sc as plsc

N = 10000
E = 160000
BATCH = 8
INPUT_DIM = 2
UNITS = 64
IN_SIZE = INPUT_DIM + UNITS
NUM_MAT = 5
ROW = BATCH * IN_SIZE

NBLK = 5
DB = 128
ROWP = NBLK * DB
NP = 10240
NSC = 2
NTEC = 16
CH = 128
NCHP = 88
MAXE = NCHP * CH
RPT = NP // NTEC

BM = 800


def _accum_chunk(acc_t, rows_b, ebuf_e, nv):
    def group(g, _):
        w_v = plsc.bitcast(ebuf_e[2, pl.ds(g * 16, 16)], jnp.float32)
        dst_v = ebuf_e[1, pl.ds(g * 16, 16)]
        for l in range(16):
            w = w_v[l]
            d = dst_v[l]
            e = g * 16 + l
            for k in range(nv):
                sl = pl.ds(k * 16, 16)
                acc_t[d, sl] = acc_t[d, sl] + rows_b[e, sl] * w
        return 0
    lax.fori_loop(0, CH // 16, group, 0)


def _spmm_body(xb, ep, out,
               eb0, eb1, eb2, eb3, rows0, rows1, acc_t,
               es0, es1, es2, es3, gs0, gs1):
    s = lax.axis_index("s")
    c = lax.axis_index("c")
    rbase = RPT * s

    ebuf = (eb0, eb1, eb2, eb3)
    esem = (es0, es1, es2, es3)
    rows = (rows0, rows1)
    gsem = (gs0, gs1)

    def edata_issue(blk, jc, e):
        pltpu.async_copy(ep.at[blk, s, jc], ebuf[e], esem[e])

    def edata_wait(blk, jc, e):
        pltpu.make_async_copy(ep.at[blk, s, jc], ebuf[e], esem[e]).wait()

    def gather_issue(e, rb):
        pltpu.async_copy(xb.at[ebuf[e].at[0]], rows[rb], gsem[rb])

    def gather_wait(e, rb):
        pltpu.make_async_copy(xb.at[ebuf[e].at[0]], rows[rb], gsem[rb]).wait()

    def chunk_iter(blk, jc, e, rb, nv):
        gather_wait(e, rb)

        @pl.when(jc + 1 < NCHP)
        def _g():
            edata_wait(blk, jc + 1, (e + 1) % 4)
            gather_issue((e + 1) % 4, rb ^ 1)

        @pl.when(jc + 2 < NCHP)
        def _e():
            edata_issue(blk, jc + 2, (e + 2) % 4)
        _accum_chunk(acc_t, rows[rb], ebuf[e], nv)

    def do_block(blk, nv):
        zeros = jnp.zeros((16,), jnp.float32)

        def zrow(r, _):
            for k in range(DB // 16):
                acc_t[r, pl.ds(k * 16, 16)] = zeros
            return 0
        lax.fori_loop(0, RPT, zrow, 0)

        edata_issue(blk, 0, 0)
        edata_issue(blk, 1, 1)
        edata_wait(blk, 0, 0)
        gather_issue(0, 0)

        def quad(jj, _):
            jc = 4 * jj
            for b in range(4):
                chunk_iter(blk, jc + b, b, b % 2, nv)
            return 0
        lax.fori_loop(0, NCHP // 4, quad, 0)

        pltpu.sync_copy(acc_t, out.at[blk, pl.ds(rbase, RPT)])

    def full_block(i, _):
        do_block(2 * i + c, DB // 16)
        return 0
    lax.fori_loop(0, 2, full_block, 0)

    @pl.when(c == 0)
    def _tail():
        do_block(4, 1)


def _spmm_sc(xb_flat, ep):
    mesh = plsc.VectorSubcoreMesh(core_axis_name="c", subcore_axis_name="s")
    scratch = [
        pltpu.VMEM((3, CH), jnp.int32),
        pltpu.VMEM((3, CH), jnp.int32),
        pltpu.VMEM((3, CH), jnp.int32),
        pltpu.VMEM((3, CH), jnp.int32),
        pltpu.VMEM((CH, DB), jnp.float32),
        pltpu.VMEM((CH, DB), jnp.float32),
        pltpu.VMEM((RPT, DB), jnp.float32),
        pltpu.SemaphoreType.DMA,
        pltpu.SemaphoreType.DMA,
        pltpu.SemaphoreType.DMA,
        pltpu.SemaphoreType.DMA,
        pltpu.SemaphoreType.DMA,
        pltpu.SemaphoreType.DMA,
    ]
    fn = pl.kernel(
        _spmm_body,
        out_type=jax.ShapeDtypeStruct((NBLK, NP, DB), jnp.float32),
        mesh=mesh,
        scratch_types=scratch,
        compiler_params=pltpu.CompilerParams(needs_layout_passes=False),
    )
    return fn(xb_flat, ep)


def _gate_body(x_ref, w_ref, b_ref, hx_ref, state2_ref, u_ref):
    v = jnp.dot(x_ref[...], w_ref[...], preferred_element_type=jnp.float32)
    v = jax.nn.sigmoid(v + b_ref[...])
    r = v[:, :UNITS]
    u = v[:, UNITS:]
    state2_ref[...] = r * hx_ref[...]
    u_ref[...] = u


def _cand_body(x_ref, w_ref, b_ref, hx_ref, u_ref, out_ref):
    v = jnp.dot(x_ref[...], w_ref[...], preferred_element_type=jnp.float32)
    c = jnp.tanh(v + b_ref[...])
    u = u_ref[...]
    out_ref[...] = u * hx_ref[...] + (1.0 - u) * c


def _gate_call(x5, Wg_r, bg, hxr):
    grid = (BATCH * N // BM,)
    return pl.pallas_call(
        _gate_body,
        grid=grid,
        in_specs=[
            pl.BlockSpec((BM, NUM_MAT * IN_SIZE), lambda i: (i, 0)),
            pl.BlockSpec((NUM_MAT * IN_SIZE, 2 * UNITS), lambda i: (0, 0)),
            pl.BlockSpec((1, 2 * UNITS), lambda i: (0, 0)),
            pl.BlockSpec((BM, UNITS), lambda i: (i, 0)),
        ],
        out_specs=[
            pl.BlockSpec((BM, UNITS), lambda i: (i, 0)),
            pl.BlockSpec((BM, UNITS), lambda i: (i, 0)),
        ],
        out_shape=[
            jax.ShapeDtypeStruct((BATCH * N, UNITS), jnp.float32),
            jax.ShapeDtypeStruct((BATCH * N, UNITS), jnp.float32),
        ],
    )(x5, Wg_r, bg.reshape(1, -1), hxr)


def _cand_call(x5, Wc_r, bc, hxr, u):
    grid = (BATCH * N // BM,)
    return pl.pallas_call(
        _cand_body,
        grid=grid,
        in_specs=[
            pl.BlockSpec((BM, NUM_MAT * IN_SIZE), lambda i: (i, 0)),
            pl.BlockSpec((NUM_MAT * IN_SIZE, UNITS), lambda i: (0, 0)),
            pl.BlockSpec((1, UNITS), lambda i: (0, 0)),
            pl.BlockSpec((BM, UNITS), lambda i: (i, 0)),
            pl.BlockSpec((BM, UNITS), lambda i: (i, 0)),
        ],
        out_specs=pl.BlockSpec((BM, UNITS), lambda i: (i, 0)),
        out_shape=jax.ShapeDtypeStruct((BATCH * N, UNITS), jnp.float32),
    )(x5, Wc_r, bc.reshape(1, -1), hxr, u)


def _pack_edges(src_, dst, w):
    tid = dst // RPT
    order = jnp.argsort(tid, stable=True)
    tid_s = tid[order]
    src_s = src_[order]
    dstr_s = dst[order] - tid_s * RPT
    w_s = w[order]
    offs = jnp.concatenate([jnp.zeros((1,), jnp.int32),
                            jnp.cumsum(jnp.bincount(tid, length=NTEC))
                            .astype(jnp.int32)[:-1]])
    pos = tid_s * MAXE + (jnp.arange(E, dtype=jnp.int32) - offs[tid_s])
    srcp = jnp.zeros((NTEC * MAXE,), jnp.int32).at[pos].set(src_s)
    dstp = jnp.zeros((NTEC * MAXE,), jnp.int32).at[pos].set(dstr_s)
    wp = jnp.zeros((NTEC * MAXE,), jnp.int32).at[pos].set(
        lax.bitcast_convert_type(w_s, jnp.int32))
    srcp = srcp.reshape(1, NTEC, NCHP, 1, CH)
    dstp = dstp.reshape(1, NTEC, NCHP, 1, CH)
    wp = wp.reshape(1, NTEC, NCHP, 1, CH)
    offs_b = (jnp.arange(NBLK, dtype=jnp.int32) * NP).reshape(NBLK, 1, 1, 1, 1)
    srcp = srcp + offs_b
    dstp = jnp.broadcast_to(dstp, (NBLK, NTEC, NCHP, 1, CH))
    wp = jnp.broadcast_to(wp, (NBLK, NTEC, NCHP, 1, CH))
    return jnp.concatenate([srcp, dstp, wp], axis=3)


def _to_blocked(x):
    xp = jnp.pad(x, ((0, NP - N), (0, ROWP - ROW)))
    return xp.reshape(NP, NBLK, DB).transpose(1, 0, 2)


def _from_blocked(xb):
    x = xb.transpose(1, 0, 2).reshape(NP, ROWP)[:N, :ROW]
    return x.reshape(BATCH * N, IN_SIZE)


def _diffusion_sc(xb, ep0, ep1):
    a1 = _spmm_sc(xb.reshape(NBLK * NP, DB), ep0)
    a2 = _spmm_sc(a1.reshape(NBLK * NP, DB), ep0)
    a3 = _spmm_sc(a1.reshape(NBLK * NP, DB), ep1)
    a4 = _spmm_sc(a3.reshape(NBLK * NP, DB), ep1)
    return a1, a2, a3, a4


def _fold_w(W, out_dim):
    Wt = W.reshape(IN_SIZE, NUM_MAT, out_dim).transpose(1, 0, 2)
    Wf = jnp.stack([
        Wt[0] - Wt[2],
        Wt[1] - Wt[4],
        2.0 * Wt[2],
        Wt[3],
        2.0 * Wt[4],
    ])
    return Wf.reshape(NUM_MAT * IN_SIZE, out_dim)


def kernel(inputs, hx, src0, dst0, w0, src1, dst1, w1, Wg, bg, Wc, bc):
    it = jnp.transpose(inputs.reshape(BATCH, N, INPUT_DIM), (1, 0, 2))
    hxt = jnp.transpose(hx.reshape(BATCH, N, UNITS), (1, 0, 2))
    hxr = hxt.reshape(BATCH * N, UNITS)
    x0 = jnp.concatenate([it, hxt], axis=2).reshape(N, ROW)

    ep0 = _pack_edges(src0, dst0, w0)
    ep1 = _pack_edges(src1, dst1, w1)

    Wg_r = _fold_w(Wg, 2 * UNITS)
    Wc_r = _fold_w(Wc, UNITS)

    xb = _to_blocked(x0)
    ms = _diffusion_sc(xb, ep0, ep1)
    x5 = jnp.concatenate(
        [x0.reshape(BATCH * N, IN_SIZE)] + [_from_blocked(m) for m in ms],
        axis=1)
    state2, u = _gate_call(x5, Wg_r, bg, hxr)

    x0c = jnp.concatenate(
        [it, state2.reshape(N, BATCH, UNITS)], axis=2).reshape(N, ROW)
    xbc = _to_blocked(x0c)
    msc = _diffusion_sc(xbc, ep0, ep1)
    x5c = jnp.concatenate(
        [x0c.reshape(BATCH * N, IN_SIZE)] + [_from_blocked(m) for m in msc],
        axis=1)
    new = _cand_call(x5c, Wc_r, bc, hxr, u)

    return jnp.transpose(new.reshape(N, BATCH, UNITS), (1, 0, 2)).reshape(
        BATCH, N * UNITS)


# speedup vs baseline: 1.1872x; 1.1872x over previous
import functools

import jax
import jax.numpy as jnp
from jax import lax
from jax.experimental import pallas as pl
from jax.experimental.pallas import tpu as pltpu
from jax.experimental.pallas import tpu_sc as plsc

N = 10000
E = 160000
BATCH = 8
INPUT_DIM = 2
UNITS = 64
IN_SIZE = INPUT_DIM + UNITS
NUM_MAT = 5
ROW = BATCH * IN_SIZE

NBLK = 5
DB = 128
ROWP = NBLK * DB
NP = 10240
NSC = 2
NTEC = 16
CH = 128
NCHP = 88
MAXE = NCHP * CH
RPT = NP // NTEC

BM = 800


def _scale_chunk(rows_b, ebuf_e, nv):
    def group(g, _):
        w_v = plsc.bitcast(ebuf_e[2, pl.ds(g * 16, 16)], jnp.float32)
        for l in range(16):
            w = w_v[l]
            e = g * 16 + l
            for k in range(nv):
                sl = pl.ds(k * 16, 16)
                rows_b[e, sl] = rows_b[e, sl] * w
        return 0
    lax.fori_loop(0, CH // 16, group, 0)


def _spmm_body(xb, ep, out,
               eb0, eb1, eb2, eb3, rows0, rows1, acc,
               es0, es1, es2, es3, gs0, gs1, ss0, ss1):
    s = lax.axis_index("s")
    c = lax.axis_index("c")
    rbase = RPT * s

    ebuf = (eb0, eb1, eb2, eb3)
    esem = (es0, es1, es2, es3)
    rows = (rows0, rows1)
    gsem = (gs0, gs1)
    ssem = (ss0, ss1)

    def edata_issue(blk, jc, e):
        pltpu.async_copy(ep.at[blk, s, jc], ebuf[e], esem[e])

    def edata_wait(blk, jc, e):
        pltpu.make_async_copy(ep.at[blk, s, jc], ebuf[e], esem[e]).wait()

    def gather_issue(e, rb):
        pltpu.async_copy(xb.at[ebuf[e].at[0]], rows[rb], gsem[rb])

    def gather_wait(e, rb):
        pltpu.make_async_copy(xb.at[ebuf[e].at[0]], rows[rb], gsem[rb]).wait()

    def scatter_issue(e, rb):
        pltpu.async_copy(rows[rb], acc.at[ebuf[e].at[1]], ssem[rb],
                         add=True)

    def scatter_wait(e, rb):
        pltpu.make_async_copy(rows[rb], acc.at[ebuf[e].at[1]],
                              ssem[rb]).wait()

    def chunk_iter(blk, jc, e, rb, nv):
        gather_wait(e, rb)

        @pl.when(jc + 1 < NCHP)
        def _g():
            scatter_wait(e, rb ^ 1)
            edata_wait(blk, jc + 1, (e + 1) % 4)
            gather_issue((e + 1) % 4, rb ^ 1)

        @pl.when(jc + 2 < NCHP)
        def _e():
            edata_issue(blk, jc + 2, (e + 2) % 4)
        _scale_chunk(rows[rb], ebuf[e], nv)
        scatter_issue(e, rb)

    def do_block(blk, nv):
        zeros = jnp.zeros((16,), jnp.float32)

        def zrow(r, _):
            for k in range(DB // 16):
                rows0[r, pl.ds(k * 16, 16)] = zeros
            return 0
        lax.fori_loop(0, CH, zrow, 0)
        for q in range(RPT // CH):
            pltpu.async_copy(rows0, acc.at[pl.ds(rbase + q * CH, CH)],
                             gsem[q % 2])
        for q in range(RPT // CH):
            pltpu.make_async_copy(rows0, acc.at[pl.ds(rbase, CH)],
                                  gsem[q % 2]).wait()

        edata_issue(blk, 0, 0)
        edata_issue(blk, 1, 1)
        pltpu.async_copy(rows0, out.at[blk, pl.ds(rbase, CH)], ss0)
        pltpu.async_copy(rows1, out.at[blk, pl.ds(rbase, CH)], ss1)
        edata_wait(blk, 0, 0)
        gather_issue(0, 0)

        def quad(jj, _):
            jc = 4 * jj
            for b in range(4):
                chunk_iter(blk, jc + b, b, b % 2, nv)
            return 0
        lax.fori_loop(0, NCHP // 4, quad, 0)

        scatter_wait(2, 0)
        scatter_wait(2, 0)
        scatter_wait(3, 1)

        pltpu.sync_copy(acc.at[pl.ds(rbase, RPT)],
                        out.at[blk, pl.ds(rbase, RPT)])

    def full_block(i, _):
        do_block(2 * i + c, DB // 16)
        return 0
    lax.fori_loop(0, 2, full_block, 0)

    @pl.when(c == 0)
    def _tail():
        do_block(4, 1)


def _spmm_sc(xb_flat, ep):
    mesh = plsc.VectorSubcoreMesh(core_axis_name="c", subcore_axis_name="s")
    scratch = [
        pltpu.VMEM((3, CH), jnp.int32),
        pltpu.VMEM((3, CH), jnp.int32),
        pltpu.VMEM((3, CH), jnp.int32),
        pltpu.VMEM((3, CH), jnp.int32),
        pltpu.VMEM((CH, DB), jnp.float32),
        pltpu.VMEM((CH, DB), jnp.float32),
        pltpu.VMEM_SHARED((NP, DB), jnp.float32),
        pltpu.SemaphoreType.DMA,
        pltpu.SemaphoreType.DMA,
        pltpu.SemaphoreType.DMA,
        pltpu.SemaphoreType.DMA,
        pltpu.SemaphoreType.DMA,
        pltpu.SemaphoreType.DMA,
        pltpu.SemaphoreType.DMA,
        pltpu.SemaphoreType.DMA,
    ]
    fn = pl.kernel(
        _spmm_body,
        out_type=jax.ShapeDtypeStruct((NBLK, NP, DB), jnp.float32),
        mesh=mesh,
        scratch_types=scratch,
        compiler_params=pltpu.CompilerParams(needs_layout_passes=False),
    )
    return fn(xb_flat, ep)


def _gate_body(x_ref, w_ref, b_ref, hx_ref, state2_ref, u_ref):
    v = jnp.dot(x_ref[...], w_ref[...], preferred_element_type=jnp.float32)
    v = jax.nn.sigmoid(v + b_ref[...])
    r = v[:, :UNITS]
    u = v[:, UNITS:]
    state2_ref[...] = r * hx_ref[...]
    u_ref[...] = u


def _cand_body(x_ref, w_ref, b_ref, hx_ref, u_ref, out_ref):
    v = jnp.dot(x_ref[...], w_ref[...], preferred_element_type=jnp.float32)
    c = jnp.tanh(v + b_ref[...])
    u = u_ref[...]
    out_ref[...] = u * hx_ref[...] + (1.0 - u) * c


def _gate_call(x5, Wg_r, bg, hxr):
    grid = (BATCH * N // BM,)
    return pl.pallas_call(
        _gate_body,
        grid=grid,
        in_specs=[
            pl.BlockSpec((BM, NUM_MAT * IN_SIZE), lambda i: (i, 0)),
            pl.BlockSpec((NUM_MAT * IN_SIZE, 2 * UNITS), lambda i: (0, 0)),
            pl.BlockSpec((1, 2 * UNITS), lambda i: (0, 0)),
            pl.BlockSpec((BM, UNITS), lambda i: (i, 0)),
        ],
        out_specs=[
            pl.BlockSpec((BM, UNITS), lambda i: (i, 0)),
            pl.BlockSpec((BM, UNITS), lambda i: (i, 0)),
        ],
        out_shape=[
            jax.ShapeDtypeStruct((BATCH * N, UNITS), jnp.float32),
            jax.ShapeDtypeStruct((BATCH * N, UNITS), jnp.float32),
        ],
    )(x5, Wg_r, bg.reshape(1, -1), hxr)


def _cand_call(x5, Wc_r, bc, hxr, u):
    grid = (BATCH * N // BM,)
    return pl.pallas_call(
        _cand_body,
        grid=grid,
        in_specs=[
            pl.BlockSpec((BM, NUM_MAT * IN_SIZE), lambda i: (i, 0)),
            pl.BlockSpec((NUM_MAT * IN_SIZE, UNITS), lambda i: (0, 0)),
            pl.BlockSpec((1, UNITS), lambda i: (0, 0)),
            pl.BlockSpec((BM, UNITS), lambda i: (i, 0)),
            pl.BlockSpec((BM, UNITS), lambda i: (i, 0)),
        ],
        out_specs=pl.BlockSpec((BM, UNITS), lambda i: (i, 0)),
        out_shape=jax.ShapeDtypeStruct((BATCH * N, UNITS), jnp.float32),
    )(x5, Wc_r, bc.reshape(1, -1), hxr, u)


def _pack_edges(src_, dst, w):
    tid = dst // RPT
    order = jnp.argsort(tid, stable=True)
    tid_s = tid[order]
    src_s = src_[order]
    dstr_s = dst[order]
    w_s = w[order]
    offs = jnp.concatenate([jnp.zeros((1,), jnp.int32),
                            jnp.cumsum(jnp.bincount(tid, length=NTEC))
                            .astype(jnp.int32)[:-1]])
    pos = tid_s * MAXE + (jnp.arange(E, dtype=jnp.int32) - offs[tid_s])
    srcp = jnp.zeros((NTEC * MAXE,), jnp.int32).at[pos].set(src_s)
    dstp = jnp.zeros((NTEC * MAXE,), jnp.int32).at[pos].set(dstr_s)
    wp = jnp.zeros((NTEC * MAXE,), jnp.int32).at[pos].set(
        lax.bitcast_convert_type(w_s, jnp.int32))
    srcp = srcp.reshape(1, NTEC, NCHP, 1, CH)
    dstp = dstp.reshape(1, NTEC, NCHP, 1, CH)
    wp = wp.reshape(1, NTEC, NCHP, 1, CH)
    offs_b = (jnp.arange(NBLK, dtype=jnp.int32) * NP).reshape(NBLK, 1, 1, 1, 1)
    srcp = srcp + offs_b
    dstp = jnp.broadcast_to(dstp, (NBLK, NTEC, NCHP, 1, CH))
    wp = jnp.broadcast_to(wp, (NBLK, NTEC, NCHP, 1, CH))
    return jnp.concatenate([srcp, dstp, wp], axis=3)


def _to_blocked(x):
    xp = jnp.pad(x, ((0, NP - N), (0, ROWP - ROW)))
    return xp.reshape(NP, NBLK, DB).transpose(1, 0, 2)


def _from_blocked(xb):
    x = xb.transpose(1, 0, 2).reshape(NP, ROWP)[:N, :ROW]
    return x.reshape(BATCH * N, IN_SIZE)


def _diffusion_sc(xb, ep0, ep1):
    a1 = _spmm_sc(xb.reshape(NBLK * NP, DB), ep0)
    a2 = _spmm_sc(a1.reshape(NBLK * NP, DB), ep0)
    a3 = _spmm_sc(a1.reshape(NBLK * NP, DB), ep1)
    a4 = _spmm_sc(a3.reshape(NBLK * NP, DB), ep1)
    return a1, a2, a3, a4


def _fold_w(W, out_dim):
    Wt = W.reshape(IN_SIZE, NUM_MAT, out_dim).transpose(1, 0, 2)
    Wf = jnp.stack([
        Wt[0] - Wt[2],
        Wt[1] - Wt[4],
        2.0 * Wt[2],
        Wt[3],
        2.0 * Wt[4],
    ])
    return Wf.reshape(NUM_MAT * IN_SIZE, out_dim)


def kernel(inputs, hx, src0, dst0, w0, src1, dst1, w1, Wg, bg, Wc, bc):
    it = jnp.transpose(inputs.reshape(BATCH, N, INPUT_DIM), (1, 0, 2))
    hxt = jnp.transpose(hx.reshape(BATCH, N, UNITS), (1, 0, 2))
    hxr = hxt.reshape(BATCH * N, UNITS)
    x0 = jnp.concatenate([it, hxt], axis=2).reshape(N, ROW)

    ep0 = _pack_edges(src0, dst0, w0)
    ep1 = _pack_edges(src1, dst1, w1)

    Wg_r = _fold_w(Wg, 2 * UNITS)
    Wc_r = _fold_w(Wc, UNITS)

    xb = _to_blocked(x0)
    ms = _diffusion_sc(xb, ep0, ep1)
    x5 = jnp.concatenate(
        [x0.reshape(BATCH * N, IN_SIZE)] + [_from_blocked(m) for m in ms],
        axis=1)
    state2, u = _gate_call(x5, Wg_r, bg, hxr)

    x0c = jnp.concatenate(
        [it, state2.reshape(N, BATCH, UNITS)], axis=2).reshape(N, ROW)
    xbc = _to_blocked(x0c)
    msc = _diffusion_sc(xbc, ep0, ep1)
    x5c = jnp.concatenate(
        [x0c.reshape(BATCH * N, IN_SIZE)] + [_from_blocked(m) for m in msc],
        axis=1)
    new = _cand_call(x5c, Wc_r, bc, hxr, u)

    return jnp.transpose(new.reshape(N, BATCH, UNITS), (1, 0, 2)).reshape(
        BATCH, N * UNITS)
